# Initial kernel scaffold; baseline (speedup 1.0000x reference)
import jax
import jax.numpy as jnp
from jax.experimental import pallas as pl


def kernel(x, table, pos_table):
    raise NotImplementedError("write your pallas kernel here")


# speedup vs baseline: 5.9875x; 5.9875x over previous
import functools

import jax
import jax.numpy as jnp
from jax import lax
from jax.experimental import pallas as pl
from jax.experimental.pallas import tpu as pltpu
from jax.experimental.pallas import tpu_sc as plsc

EMBED_DIM = 64
SEQ = 200
LANES = 16


def _build(B, S, D):
    NC, NS = 2, 16
    NW = NC * NS
    assert B % NW == 0
    rows_per_w = B // NW

    mesh = plsc.VectorSubcoreMesh(core_axis_name="c", subcore_axis_name="s")

    @functools.partial(
        pl.kernel,
        out_type=jax.ShapeDtypeStruct((B * S, D), jnp.float32),
        mesh=mesh,
        compiler_params=pltpu.CompilerParams(use_tc_tiling_on_sc=False),
        scratch_types=[
            pltpu.VMEM((S, D), jnp.float32),
            pltpu.VMEM((S,), jnp.int32),
            pltpu.VMEM((S, D), jnp.float32),
            pltpu.SemaphoreType.DMA,
        ],
    )
    def emb(xf_hbm, table_hbm, pos_hbm, out_hbm, pos_v, idx_v, rows_v, sem):
        wid = lax.axis_index("s") * NC + lax.axis_index("c")
        base = wid * rows_per_w
        pltpu.sync_copy(pos_hbm, pos_v)

        def chunk(c, carry):
            r0 = (base + c) * S
            pltpu.sync_copy(xf_hbm.at[pl.ds(r0, S)], idx_v)
            cp1 = pltpu.async_copy(
                table_hbm.at[idx_v.at[pl.ds(0, 104)]],
                rows_v.at[pl.ds(0, 104)], sem)
            cp2 = pltpu.async_copy(
                table_hbm.at[idx_v.at[pl.ds(104, 96)]],
                rows_v.at[pl.ds(104, 96)], sem)
            cp1.wait()
            cp2.wait()

            def add_row(i, carry2):
                for j in range(D // LANES):
                    sl = pl.ds(j * LANES, LANES)
                    rows_v[i, sl] = rows_v[i, sl] + pos_v[i, sl]
                return carry2

            lax.fori_loop(0, S, add_row, 0)
            pltpu.sync_copy(rows_v, out_hbm.at[pl.ds(r0, S)])
            return carry

        lax.fori_loop(0, rows_per_w, chunk, 0)

    return emb


def kernel(x, table, pos_table):
    B, S = x.shape
    D = table.shape[1]
    xf = x.reshape(-1).astype(jnp.int32)
    emb = _build(B, S, D)
    out = emb(xf, table, pos_table.astype(jnp.float32))
    return out.reshape(B, S, D)


# speedup vs baseline: 7.5562x; 1.2620x over previous
import functools

import jax
import jax.numpy as jnp
from jax import lax
from jax.experimental import pallas as pl
from jax.experimental.pallas import tpu as pltpu
from jax.experimental.pallas import tpu_sc as plsc

LANES = 16


def _build(B, S, D):
    NC, NS = 2, 16
    NW = NC * NS
    assert B % NW == 0
    rows_per_w = B // NW
    S0 = (S // 2 + 7) // 8 * 8
    S1 = S - S0

    mesh = plsc.VectorSubcoreMesh(core_axis_name="c", subcore_axis_name="s")

    @functools.partial(
        pl.kernel,
        out_type=jax.ShapeDtypeStruct((B * S, D), jnp.float32),
        mesh=mesh,
        compiler_params=pltpu.CompilerParams(use_tc_tiling_on_sc=False),
        scratch_types=[
            pltpu.VMEM((S, D), jnp.float32),
            pltpu.VMEM((S,), jnp.int32),
            pltpu.VMEM((S,), jnp.int32),
            pltpu.VMEM((S, D), jnp.float32),
            pltpu.VMEM((S, D), jnp.float32),
            pltpu.VMEM((S, D), jnp.float32),
            pltpu.VMEM((S, D), jnp.float32),
            pltpu.SemaphoreType.DMA,
            pltpu.SemaphoreType.DMA,
            pltpu.SemaphoreType.DMA,
            pltpu.SemaphoreType.DMA,
        ],
    )
    def emb(xf_hbm, table_hbm, pos_hbm, out_hbm, pos_v,
            idx0, idx1, rin0, rin1, rout0, rout1,
            gsem0, gsem1, osem0, osem1):
        wid = lax.axis_index("s") * NC + lax.axis_index("c")
        base = wid * rows_per_w
        pltpu.sync_copy(pos_hbm, pos_v)

        idxs = (idx0, idx1)
        rins = (rin0, rin1)
        routs = (rout0, rout1)
        gsems = (gsem0, gsem1)
        osems = (osem0, osem1)

        def fire_gather(c, p):
            r0 = (base + c) * S
            pltpu.sync_copy(xf_hbm.at[pl.ds(r0, S)], idxs[p])
            pltpu.async_copy(table_hbm.at[idxs[p].at[pl.ds(0, S0)]],
                             rins[p].at[pl.ds(0, S0)], gsems[p])
            pltpu.async_copy(table_hbm.at[idxs[p].at[pl.ds(S0, S1)]],
                             rins[p].at[pl.ds(S0, S1)], gsems[p])

        def wait_gather(p):
            pltpu.make_async_copy(table_hbm.at[idxs[p].at[pl.ds(0, S0)]],
                                  rins[p].at[pl.ds(0, S0)], gsems[p]).wait()
            pltpu.make_async_copy(table_hbm.at[idxs[p].at[pl.ds(S0, S1)]],
                                  rins[p].at[pl.ds(S0, S1)], gsems[p]).wait()

        def wait_store(p):
            pltpu.make_async_copy(routs[p], out_hbm.at[pl.ds(0, S)],
                                  osems[p]).wait()

        fire_gather(0, 0)

        def group(g, carry):
            for p in range(2):
                c = 2 * g + p

                @pl.when(c < rows_per_w - 1)
                def _():
                    fire_gather(c + 1, 1 - p)

                wait_gather(p)

                @pl.when(c >= 2)
                def _():
                    wait_store(p)

                rin, rout = rins[p], routs[p]

                @plsc.parallel_loop(0, S, unroll=4)
                def _(i):
                    for j in range(D // LANES):
                        sl = pl.ds(j * LANES, LANES)
                        rout[i, sl] = rin[i, sl] + pos_v[i, sl]

                pltpu.async_copy(rout, out_hbm.at[pl.ds((base + c) * S, S)],
                                 osems[p])
            return carry

        lax.fori_loop(0, rows_per_w // 2, group, 0)
        wait_store(0)
        wait_store(1)

    return emb


def kernel(x, table, pos_table):
    B, S = x.shape
    D = table.shape[1]
    xf = x.reshape(-1).astype(jnp.int32)
    emb = _build(B, S, D)
    out = emb(xf, table, pos_table.astype(jnp.float32))
    return out.reshape(B, S, D)
